# baseline (speedup 1.0000x reference)
import jax
import jax.numpy as jnp
from jax import lax
from jax.experimental import pallas as pl
from jax.experimental.pallas import tpu as pltpu
from jax.experimental.pallas import tpu_sc as plsc

N = 10000
E = 320000
D = 128
H = 16
C = 7

NC = 2
NS = 16
NW = NC * NS
RPT = N // NS
EW = E // NW
CHUNK = 128
NCHUNK = EW // CHUNK
TAIL = EW - NCHUNK * CHUNK
NBUF = 4
PREF = 2
ZR = 128
NBLK = RPT // 16

_mesh = plsc.VectorSubcoreMesh(
    core_axis_name="c", subcore_axis_name="s", num_cores=NC, num_subcores=NS
)
_sc_params = pltpu.CompilerParams(
    needs_layout_passes=False, use_tc_tiling_on_sc=False)


def _worker_id():
    return lax.axis_index("c") * NS + lax.axis_index("s")


def _deg_body(edge_hbm, out_hbm, dst_v, deg_v):
    wid = _worker_id()
    pltpu.sync_copy(edge_hbm.at[1, pl.ds(wid * EW, EW)], dst_v)
    zero = jnp.zeros((16,), jnp.float32)

    def zbody(i, carry):
        deg_v[pl.ds(i * 16, 16)] = zero
        return carry

    lax.fori_loop(0, N // 16, zbody, 0)
    ones = jnp.full((16,), 1.0, jnp.float32)

    def body(i, carry):
        for u in range(5):
            idx = dst_v[pl.ds(i * 80 + u * 16, 16)]
            plsc.addupdate_scatter(deg_v, [idx], ones)
        return carry

    lax.fori_loop(0, EW // 80, body, 0)
    pltpu.sync_copy(deg_v, out_hbm.at[wid])


@jax.jit
def _deg_kernel(edge_index):
    return pl.kernel(
        _deg_body,
        out_type=jax.ShapeDtypeStruct((NW, N), jnp.float32),
        mesh=_mesh,
        scratch_types=[
            pltpu.VMEM((EW,), jnp.int32),
            pltpu.VMEM((N,), jnp.float32),
        ],
        compiler_params=_sc_params,
    )(edge_index)


def _zero_acc(zbuf_v, acc_sp, r0):
    zero = jnp.zeros((16,), jnp.float32)

    def zbody(i, carry):
        zbuf_v[i] = zero
        return carry

    lax.fori_loop(0, ZR, zbody, 0)
    for q in range(4):
        pltpu.sync_copy(zbuf_v, acc_sp.at[pl.ds(r0 + q * ZR, ZR)])
    pltpu.sync_copy(zbuf_v.at[pl.ds(0, RPT - 4 * ZR)],
                    acc_sp.at[pl.ds(r0 + 4 * ZR, RPT - 4 * ZR)])


def _edge_loop(src_v, dst_v, rows_v, y_sp, acc_sp, gsem, ssem):
    def gather(j, b):
        pltpu.async_copy(y_sp.at[src_v.at[pl.ds(j * CHUNK, CHUNK)]],
                         rows_v.at[b], gsem.at[b])

    def scatter(j, b):
        pltpu.async_copy(rows_v.at[b],
                         acc_sp.at[dst_v.at[pl.ds(j * CHUNK, CHUNK)]],
                         ssem.at[b], add=True)

    def wait_gather(j, b):
        pltpu.make_async_copy(y_sp.at[src_v.at[pl.ds(j * CHUNK, CHUNK)]],
                              rows_v.at[b], gsem.at[b]).wait()

    def wait_scatter(j, b):
        pltpu.make_async_copy(rows_v.at[b],
                              acc_sp.at[dst_v.at[pl.ds(j * CHUNK, CHUNK)]],
                              ssem.at[b]).wait()

    for jp in range(PREF):
        gather(jp, jp % NBUF)

    def body(j, carry):
        b = lax.rem(j, NBUF)
        wait_gather(j, b)
        scatter(j, b)
        jn = j + PREF
        bn = lax.rem(jn, NBUF)

        @pl.when(jn < NCHUNK)
        def _():
            @pl.when(jn >= NBUF)
            def _():
                wait_scatter(jn - NBUF, bn)
            gather(jn, bn)

        return carry

    lax.fori_loop(0, NCHUNK, body, 0)
    for j in range(NCHUNK - NBUF, NCHUNK):
        wait_scatter(j, j % NBUF)
    t0 = NCHUNK * CHUNK
    pltpu.async_copy(y_sp.at[src_v.at[pl.ds(t0, TAIL)]],
                     rows_v.at[0, pl.ds(0, TAIL)], gsem.at[0])
    pltpu.make_async_copy(y_sp.at[src_v.at[pl.ds(t0, TAIL)]],
                          rows_v.at[0, pl.ds(0, TAIL)], gsem.at[0]).wait()
    pltpu.sync_copy(rows_v.at[0, pl.ds(0, TAIL)],
                    acc_sp.at[dst_v.at[pl.ds(t0, TAIL)]], add=True)


def _msg_body(y_hbm, edge_hbm, out_hbm,
              src_v, dst_v, rows_v, zbuf_v, y_sp, acc_sp, gsem, ssem):
    c = lax.axis_index("c")
    s = lax.axis_index("s")
    wid = c * NS + s
    r0 = s * RPT
    pltpu.sync_copy(y_hbm.at[pl.ds(r0, RPT)], y_sp.at[pl.ds(r0, RPT)])
    _zero_acc(zbuf_v, acc_sp, r0)
    pltpu.sync_copy(edge_hbm.at[0, pl.ds(wid * EW, EW)], src_v)
    pltpu.sync_copy(edge_hbm.at[1, pl.ds(wid * EW, EW)], dst_v)
    plsc.subcore_barrier()
    _edge_loop(src_v, dst_v, rows_v, y_sp, acc_sp, gsem, ssem)
    plsc.subcore_barrier()
    pltpu.sync_copy(acc_sp.at[pl.ds(r0, RPT)], out_hbm.at[c, pl.ds(r0, RPT)])


@jax.jit
def _msg_kernel(y, edge_index):
    return pl.kernel(
        _msg_body,
        out_type=jax.ShapeDtypeStruct((NC, N, H), jnp.float32),
        mesh=_mesh,
        scratch_types=[
            pltpu.VMEM((EW,), jnp.int32),
            pltpu.VMEM((EW,), jnp.int32),
            pltpu.VMEM((NBUF, CHUNK, H), jnp.float32),
            pltpu.VMEM((ZR, H), jnp.float32),
            pltpu.VMEM_SHARED((N, H), jnp.float32),
            pltpu.VMEM_SHARED((N, H), jnp.float32),
            pltpu.SemaphoreType.DMA((NBUF,)),
            pltpu.SemaphoreType.DMA((NBUF,)),
        ],
        compiler_params=_sc_params,
    )(y, edge_index)


def _msg2_body(acc1_hbm, y1_hbm, dinv_hbm, w2_hbm, b1_hbm, edge_hbm,
               out_hbm, y2p_hbm,
               src_v, dst_v, rows_v, zbuf_v, a0_v, a1_v, y1_v, dv_v,
               y2_v, y2p_v, w2_v, b1_v, y_sp, acc_sp, gsem, ssem):
    c = lax.axis_index("c")
    s = lax.axis_index("s")
    wid = c * NS + s
    r0 = s * RPT
    pltpu.sync_copy(acc1_hbm.at[0, pl.ds(r0, RPT)], a0_v)
    pltpu.sync_copy(acc1_hbm.at[1, pl.ds(r0, RPT)], a1_v)
    pltpu.sync_copy(y1_hbm.at[pl.ds(r0, RPT)], y1_v)
    pltpu.sync_copy(dinv_hbm.at[pl.ds(r0, RPT)], dv_v)
    pltpu.sync_copy(w2_hbm, w2_v)
    pltpu.sync_copy(b1_hbm, b1_v)
    lanes = lax.broadcasted_iota(jnp.int32, (16,), 0)
    colidx = [jnp.full((16,), cc, jnp.int32) for cc in range(H)]
    zeros16f = jnp.zeros((16,), jnp.float32)
    b1vec = b1_v[0]
    w2rows = [w2_v[k] for k in range(H)]

    def mid_blk(blk, carry):
        ridx = lanes + blk * 16
        dv = plsc.load_gather(dv_v, [ridx, colidx[0]])
        ocols = []
        for cc in range(H):
            a = (plsc.load_gather(a0_v, [ridx, colidx[cc]])
                 + plsc.load_gather(a1_v, [ridx, colidx[cc]])
                 + plsc.load_gather(y1_v, [ridx, colidx[cc]]))
            t = a * dv + b1vec[cc]
            ocols.append(jnp.maximum(t, 0.0))
        for cp in range(H):
            h2 = zeros16f
            for k in range(H):
                h2 = h2 + ocols[k] * w2rows[k][cp]
            y2c = h2 * dv
            plsc.store_scatter(y2_v, [ridx, colidx[cp]], y2c)
            plsc.store_scatter(y2p_v, [ridx, colidx[cp]], y2c * dv)
        return carry

    lax.fori_loop(0, NBLK, mid_blk, 0)
    tidx = jnp.full((16,), RPT - 1, jnp.int32)
    a = a0_v[RPT - 1] + a1_v[RPT - 1] + y1_v[RPT - 1]
    dvt = plsc.load_gather(dv_v, [tidx, colidx[0]])
    o = jnp.maximum(a * dvt + b1vec, 0.0)
    h2 = zeros16f
    for k in range(H):
        h2 = h2 + w2rows[k] * o[k]
    y2row = h2 * dvt
    y2_v[RPT - 1] = y2row
    y2p_v[RPT - 1] = y2row * dvt
    pltpu.sync_copy(y2_v, y_sp.at[pl.ds(r0, RPT)])
    pltpu.sync_copy(y2p_v, y2p_hbm.at[pl.ds(r0, RPT)])
    _zero_acc(zbuf_v, acc_sp, r0)
    pltpu.sync_copy(edge_hbm.at[0, pl.ds(wid * EW, EW)], src_v)
    pltpu.sync_copy(edge_hbm.at[1, pl.ds(wid * EW, EW)], dst_v)
    plsc.subcore_barrier()
    _edge_loop(src_v, dst_v, rows_v, y_sp, acc_sp, gsem, ssem)
    plsc.subcore_barrier()
    pltpu.sync_copy(acc_sp.at[pl.ds(r0, RPT)], a0_v)

    def scale_row(r, carry):
        dvg = plsc.load_gather(dv_v, [jnp.full((16,), r, jnp.int32), colidx[0]])
        a0_v[r] = a0_v[r] * dvg
        return carry

    lax.fori_loop(0, RPT, scale_row, 0)
    pltpu.sync_copy(a0_v, out_hbm.at[c, pl.ds(r0, RPT)])


@jax.jit
def _msg2_kernel(acc1, y1, dinv, W2p, b1r, edge_index):
    return pl.kernel(
        _msg2_body,
        out_type=[
            jax.ShapeDtypeStruct((NC, N, H), jnp.float32),
            jax.ShapeDtypeStruct((N, H), jnp.float32),
        ],
        mesh=_mesh,
        scratch_types=[
            pltpu.VMEM((EW,), jnp.int32),
            pltpu.VMEM((EW,), jnp.int32),
            pltpu.VMEM((NBUF, CHUNK, H), jnp.float32),
            pltpu.VMEM((ZR, H), jnp.float32),
            pltpu.VMEM((RPT, H), jnp.float32),
            pltpu.VMEM((RPT, H), jnp.float32),
            pltpu.VMEM((RPT, H), jnp.float32),
            pltpu.VMEM((RPT, 1), jnp.float32),
            pltpu.VMEM((RPT, H), jnp.float32),
            pltpu.VMEM((RPT, H), jnp.float32),
            pltpu.VMEM((H, H), jnp.float32),
            pltpu.VMEM((1, H), jnp.float32),
            pltpu.VMEM_SHARED((N, H), jnp.float32),
            pltpu.VMEM_SHARED((N, H), jnp.float32),
            pltpu.SemaphoreType.DMA((NBUF,)),
            pltpu.SemaphoreType.DMA((NBUF,)),
        ],
        compiler_params=_sc_params,
    )(acc1, y1, dinv, W2p, b1r, edge_index)


BN = 1024


def _prep_body(deg_ref, x_ref, w1_ref, dinv_ref, y1_ref):
    deg = jnp.sum(deg_ref[...], axis=0) + 1.0
    dinv = lax.rsqrt(deg)[:, None]
    dinv_ref[...] = dinv
    h = jnp.dot(x_ref[...], w1_ref[...], preferred_element_type=jnp.float32)
    y1_ref[...] = dinv * h


@jax.jit
def _prep_kernel(deg_parts, x, W1):
    return pl.pallas_call(
        _prep_body,
        grid=(pl.cdiv(N, BN),),
        in_specs=[
            pl.BlockSpec((NW, BN), lambda i: (0, i)),
            pl.BlockSpec((BN, D), lambda i: (i, 0)),
            pl.BlockSpec((D, H), lambda i: (0, 0)),
        ],
        out_specs=[
            pl.BlockSpec((BN, 1), lambda i: (i, 0)),
            pl.BlockSpec((BN, H), lambda i: (i, 0)),
        ],
        out_shape=[
            jax.ShapeDtypeStruct((N, 1), jnp.float32),
            jax.ShapeDtypeStruct((N, H), jnp.float32),
        ],
    )(deg_parts, x, W1)


def _fin_body(acc_ref, y2p_ref, b2_ref, out_ref):
    t = acc_ref[0] + acc_ref[1] + y2p_ref[...] + b2_ref[...]
    logits = t[:, :C]
    m = jnp.max(logits, axis=1, keepdims=True)
    ex = jnp.exp(logits - m)
    lse = jnp.log(jnp.sum(ex, axis=1, keepdims=True)) + m
    out_ref[...] = logits - lse


@jax.jit
def _fin_kernel(acc2, y2p, b2p):
    return pl.pallas_call(
        _fin_body,
        grid=(pl.cdiv(N, BN),),
        in_specs=[
            pl.BlockSpec((NC, BN, H), lambda i: (0, i, 0)),
            pl.BlockSpec((BN, H), lambda i: (i, 0)),
            pl.BlockSpec((1, H), lambda i: (0, 0)),
        ],
        out_specs=pl.BlockSpec((BN, C), lambda i: (i, 0)),
        out_shape=jax.ShapeDtypeStruct((N, C), jnp.float32),
    )(acc2, y2p, b2p)


def kernel(x, edge_index, W1, b1, W2, b2):
    W2p = jnp.pad(W2, ((0, 0), (0, H - C)))
    b1r = b1.reshape(1, H)
    b2p = jnp.pad(b2, (0, H - C)).reshape(1, H)

    deg_parts = _deg_kernel(edge_index)
    dinv, y1 = _prep_kernel(deg_parts, x, W1)
    acc1 = _msg_kernel(y1, edge_index)
    acc2, y2p = _msg2_kernel(acc1, y1, dinv, W2p, b1r, edge_index)
    return _fin_kernel(acc2, y2p, b2p)


# speedup vs baseline: 1.0909x; 1.0909x over previous
import jax
import jax.numpy as jnp
from jax import lax
from jax.experimental import pallas as pl
from jax.experimental.pallas import tpu as pltpu
from jax.experimental.pallas import tpu_sc as plsc

N = 10000
E = 320000
D = 128
H = 16
C = 7

NC = 2
NS = 16
NW = NC * NS
RPT = N // NS
EW = E // NW
CHUNK = 128
NCHUNK = EW // CHUNK
TAIL = EW - NCHUNK * CHUNK
NBUF = 4
PREF = 2
ZR = 128
NBLK = RPT // 16

_mesh = plsc.VectorSubcoreMesh(
    core_axis_name="c", subcore_axis_name="s", num_cores=NC, num_subcores=NS
)
_sc_params = pltpu.CompilerParams(
    needs_layout_passes=False, use_tc_tiling_on_sc=False)


def _worker_id():
    return lax.axis_index("c") * NS + lax.axis_index("s")


def _deg_body(edge_hbm, out_hbm, dst_v, deg_v):
    wid = _worker_id()
    pltpu.sync_copy(edge_hbm.at[1, pl.ds(wid * EW, EW)], dst_v)
    zero = jnp.zeros((16,), jnp.float32)

    def zbody(i, carry):
        deg_v[pl.ds(i * 16, 16)] = zero
        return carry

    lax.fori_loop(0, N // 16, zbody, 0)
    ones = jnp.full((16,), 1.0, jnp.float32)

    def body(i, carry):
        for u in range(5):
            idx = dst_v[pl.ds(i * 80 + u * 16, 16)]
            plsc.addupdate_scatter(deg_v, [idx], ones)
        return carry

    lax.fori_loop(0, EW // 80, body, 0)
    pltpu.sync_copy(deg_v, out_hbm.at[wid])


@jax.jit
def _deg_kernel(edge_index):
    return pl.kernel(
        _deg_body,
        out_type=jax.ShapeDtypeStruct((NW, N), jnp.float32),
        mesh=_mesh,
        scratch_types=[
            pltpu.VMEM((EW,), jnp.int32),
            pltpu.VMEM((N,), jnp.float32),
        ],
        compiler_params=_sc_params,
    )(edge_index)


def _zero_acc(zbuf_v, acc_sp, r0):
    zero = jnp.zeros((16,), jnp.float32)

    def zbody(i, carry):
        zbuf_v[i] = zero
        return carry

    lax.fori_loop(0, ZR, zbody, 0)
    for q in range(4):
        pltpu.sync_copy(zbuf_v, acc_sp.at[pl.ds(r0 + q * ZR, ZR)])
    pltpu.sync_copy(zbuf_v.at[pl.ds(0, RPT - 4 * ZR)],
                    acc_sp.at[pl.ds(r0 + 4 * ZR, RPT - 4 * ZR)])


def _edge_loop(src_v, dst_v, rows_v, y_sp, acc_sp, gsem, ssem):
    def gather(j, b):
        pltpu.async_copy(y_sp.at[src_v.at[pl.ds(j * CHUNK, CHUNK)]],
                         rows_v.at[b], gsem.at[b])

    def scatter(j, b):
        pltpu.async_copy(rows_v.at[b],
                         acc_sp.at[dst_v.at[pl.ds(j * CHUNK, CHUNK)]],
                         ssem.at[b], add=True)

    def wait_gather(j, b):
        pltpu.make_async_copy(y_sp.at[src_v.at[pl.ds(j * CHUNK, CHUNK)]],
                              rows_v.at[b], gsem.at[b]).wait()

    def wait_scatter(j, b):
        pltpu.make_async_copy(rows_v.at[b],
                              acc_sp.at[dst_v.at[pl.ds(j * CHUNK, CHUNK)]],
                              ssem.at[b]).wait()

    for jp in range(PREF):
        gather(jp, jp % NBUF)

    def body(j, carry):
        b = lax.rem(j, NBUF)
        wait_gather(j, b)
        scatter(j, b)
        jn = j + PREF
        bn = lax.rem(jn, NBUF)

        @pl.when(jn < NCHUNK)
        def _():
            @pl.when(jn >= NBUF)
            def _():
                wait_scatter(jn - NBUF, bn)
            gather(jn, bn)

        return carry

    lax.fori_loop(0, NCHUNK, body, 0)
    for j in range(NCHUNK - NBUF, NCHUNK):
        wait_scatter(j, j % NBUF)
    t0 = NCHUNK * CHUNK
    pltpu.async_copy(y_sp.at[src_v.at[pl.ds(t0, TAIL)]],
                     rows_v.at[0, pl.ds(0, TAIL)], gsem.at[0])
    pltpu.make_async_copy(y_sp.at[src_v.at[pl.ds(t0, TAIL)]],
                          rows_v.at[0, pl.ds(0, TAIL)], gsem.at[0]).wait()
    pltpu.sync_copy(rows_v.at[0, pl.ds(0, TAIL)],
                    acc_sp.at[dst_v.at[pl.ds(t0, TAIL)]], add=True)


def _msg_body(y_hbm, edge_hbm, out_hbm,
              src_v, dst_v, rows_v, zbuf_v, y_sp, acc_sp, gsem, ssem):
    c = lax.axis_index("c")
    s = lax.axis_index("s")
    wid = c * NS + s
    r0 = s * RPT
    pltpu.sync_copy(y_hbm.at[pl.ds(r0, RPT)], y_sp.at[pl.ds(r0, RPT)])
    _zero_acc(zbuf_v, acc_sp, r0)
    pltpu.sync_copy(edge_hbm.at[0, pl.ds(wid * EW, EW)], src_v)
    pltpu.sync_copy(edge_hbm.at[1, pl.ds(wid * EW, EW)], dst_v)
    plsc.subcore_barrier()
    _edge_loop(src_v, dst_v, rows_v, y_sp, acc_sp, gsem, ssem)
    plsc.subcore_barrier()
    pltpu.sync_copy(acc_sp.at[pl.ds(r0, RPT)], out_hbm.at[c, pl.ds(r0, RPT)])


@jax.jit
def _msg_kernel(y, edge_index):
    return pl.kernel(
        _msg_body,
        out_type=jax.ShapeDtypeStruct((NC, N, H), jnp.float32),
        mesh=_mesh,
        scratch_types=[
            pltpu.VMEM((EW,), jnp.int32),
            pltpu.VMEM((EW,), jnp.int32),
            pltpu.VMEM((NBUF, CHUNK, H), jnp.float32),
            pltpu.VMEM((ZR, H), jnp.float32),
            pltpu.VMEM_SHARED((N, H), jnp.float32),
            pltpu.VMEM_SHARED((N, H), jnp.float32),
            pltpu.SemaphoreType.DMA((NBUF,)),
            pltpu.SemaphoreType.DMA((NBUF,)),
        ],
        compiler_params=_sc_params,
    )(y, edge_index)


BN = 1024


def _prep_body(deg_ref, x_ref, w1_ref, dinv_ref, y1_ref):
    deg = jnp.sum(deg_ref[...], axis=0) + 1.0
    dinv = lax.rsqrt(deg)[:, None]
    dinv_ref[...] = dinv
    h = jnp.dot(x_ref[...], w1_ref[...], preferred_element_type=jnp.float32)
    y1_ref[...] = dinv * h


@jax.jit
def _prep_kernel(deg_parts, x, W1):
    return pl.pallas_call(
        _prep_body,
        grid=(pl.cdiv(N, BN),),
        in_specs=[
            pl.BlockSpec((NW, BN), lambda i: (0, i)),
            pl.BlockSpec((BN, D), lambda i: (i, 0)),
            pl.BlockSpec((D, H), lambda i: (0, 0)),
        ],
        out_specs=[
            pl.BlockSpec((BN, 1), lambda i: (i, 0)),
            pl.BlockSpec((BN, H), lambda i: (i, 0)),
        ],
        out_shape=[
            jax.ShapeDtypeStruct((N, 1), jnp.float32),
            jax.ShapeDtypeStruct((N, H), jnp.float32),
        ],
    )(deg_parts, x, W1)


def _mid_body(acc_ref, y1_ref, dinv_ref, w2_ref, b1_ref, y2_ref):
    a = acc_ref[0] + acc_ref[1] + y1_ref[...]
    dinv = dinv_ref[...]
    out1 = jnp.maximum(dinv * a + b1_ref[...], 0.0)
    h2 = jnp.dot(out1, w2_ref[...], preferred_element_type=jnp.float32)
    y2_ref[...] = dinv * h2


@jax.jit
def _mid_kernel(acc1, y1, dinv, W2p, b1r):
    return pl.pallas_call(
        _mid_body,
        grid=(pl.cdiv(N, BN),),
        in_specs=[
            pl.BlockSpec((NC, BN, H), lambda i: (0, i, 0)),
            pl.BlockSpec((BN, H), lambda i: (i, 0)),
            pl.BlockSpec((BN, 1), lambda i: (i, 0)),
            pl.BlockSpec((H, H), lambda i: (0, 0)),
            pl.BlockSpec((1, H), lambda i: (0, 0)),
        ],
        out_specs=pl.BlockSpec((BN, H), lambda i: (i, 0)),
        out_shape=jax.ShapeDtypeStruct((N, H), jnp.float32),
    )(acc1, y1, dinv, W2p, b1r)


def _fin_body(acc_ref, y2_ref, dinv_ref, b2_ref, out_ref):
    a = acc_ref[0] + acc_ref[1] + y2_ref[...]
    t = dinv_ref[...] * a + b2_ref[...]
    logits = t[:, :C]
    m = jnp.max(logits, axis=1, keepdims=True)
    ex = jnp.exp(logits - m)
    lse = jnp.log(jnp.sum(ex, axis=1, keepdims=True)) + m
    out_ref[...] = logits - lse


@jax.jit
def _fin_kernel(acc2, y2, dinv, b2p):
    return pl.pallas_call(
        _fin_body,
        grid=(pl.cdiv(N, BN),),
        in_specs=[
            pl.BlockSpec((NC, BN, H), lambda i: (0, i, 0)),
            pl.BlockSpec((BN, H), lambda i: (i, 0)),
            pl.BlockSpec((BN, 1), lambda i: (i, 0)),
            pl.BlockSpec((1, H), lambda i: (0, 0)),
        ],
        out_specs=pl.BlockSpec((BN, C), lambda i: (i, 0)),
        out_shape=jax.ShapeDtypeStruct((N, C), jnp.float32),
    )(acc2, y2, dinv, b2p)


def kernel(x, edge_index, W1, b1, W2, b2):
    W2p = jnp.pad(W2, ((0, 0), (0, H - C)))
    b1r = b1.reshape(1, H)
    b2p = jnp.pad(b2, (0, H - C)).reshape(1, H)

    deg_parts = _deg_kernel(edge_index)
    dinv, y1 = _prep_kernel(deg_parts, x, W1)
    acc1 = _msg_kernel(y1, edge_index)
    y2 = _mid_kernel(acc1, y1, dinv, W2p, b1r)
    acc2 = _msg_kernel(y2, edge_index)
    return _fin_kernel(acc2, y2, dinv, b2p)


# speedup vs baseline: 1.1642x; 1.0671x over previous
import jax
import jax.numpy as jnp
from jax import lax
from jax.experimental import pallas as pl
from jax.experimental.pallas import tpu as pltpu
from jax.experimental.pallas import tpu_sc as plsc

N = 10000
E = 320000
D = 128
H = 16
C = 7

NC = 2
NS = 16
NW = NC * NS
RPT = N // NS
EW = E // NW
CHUNK = 128
NCHUNK = EW // CHUNK
TAIL = EW - NCHUNK * CHUNK
NBUF = 6
PREF = 3
ZR = 128
NBLK = RPT // 16

_mesh = plsc.VectorSubcoreMesh(
    core_axis_name="c", subcore_axis_name="s", num_cores=NC, num_subcores=NS
)
_sc_params = pltpu.CompilerParams(
    needs_layout_passes=False, use_tc_tiling_on_sc=False)


def _worker_id():
    return lax.axis_index("c") * NS + lax.axis_index("s")


def _deg_body(edge_hbm, out_hbm, dst_v, deg_v):
    wid = _worker_id()
    pltpu.sync_copy(edge_hbm.at[1, pl.ds(wid * EW, EW)], dst_v)
    zero = jnp.zeros((16,), jnp.float32)

    def zbody(i, carry):
        deg_v[pl.ds(i * 16, 16)] = zero
        return carry

    lax.fori_loop(0, N // 16, zbody, 0)
    ones = jnp.full((16,), 1.0, jnp.float32)

    def body(i, carry):
        for u in range(5):
            idx = dst_v[pl.ds(i * 80 + u * 16, 16)]
            plsc.addupdate_scatter(deg_v, [idx], ones)
        return carry

    lax.fori_loop(0, EW // 80, body, 0)
    pltpu.sync_copy(deg_v, out_hbm.at[wid])


@jax.jit
def _deg_kernel(edge_index):
    return pl.kernel(
        _deg_body,
        out_type=jax.ShapeDtypeStruct((NW, N), jnp.float32),
        mesh=_mesh,
        scratch_types=[
            pltpu.VMEM((EW,), jnp.int32),
            pltpu.VMEM((N,), jnp.float32),
        ],
        compiler_params=_sc_params,
    )(edge_index)


def _zero_acc(zbuf_v, acc_sp, r0):
    zero = jnp.zeros((16,), jnp.float32)

    def zbody(i, carry):
        zbuf_v[i] = zero
        return carry

    lax.fori_loop(0, ZR, zbody, 0)
    for q in range(4):
        pltpu.sync_copy(zbuf_v, acc_sp.at[pl.ds(r0 + q * ZR, ZR)])
    pltpu.sync_copy(zbuf_v.at[pl.ds(0, RPT - 4 * ZR)],
                    acc_sp.at[pl.ds(r0 + 4 * ZR, RPT - 4 * ZR)])


def _edge_loop(src_v, dst_v, rows_v, y_sp, acc_sp, gsem, ssem):
    def gather(j, b):
        pltpu.async_copy(y_sp.at[src_v.at[pl.ds(j * CHUNK, CHUNK)]],
                         rows_v.at[b], gsem.at[b])

    def scatter(j, b):
        pltpu.async_copy(rows_v.at[b],
                         acc_sp.at[dst_v.at[pl.ds(j * CHUNK, CHUNK)]],
                         ssem.at[b], add=True)

    def wait_gather(j, b):
        pltpu.make_async_copy(y_sp.at[src_v.at[pl.ds(j * CHUNK, CHUNK)]],
                              rows_v.at[b], gsem.at[b]).wait()

    def wait_scatter(j, b):
        pltpu.make_async_copy(rows_v.at[b],
                              acc_sp.at[dst_v.at[pl.ds(j * CHUNK, CHUNK)]],
                              ssem.at[b]).wait()

    for jp in range(PREF):
        gather(jp, jp % NBUF)

    def body(j, carry):
        b = lax.rem(j, NBUF)
        wait_gather(j, b)
        scatter(j, b)
        jn = j + PREF
        bn = lax.rem(jn, NBUF)

        @pl.when(jn < NCHUNK)
        def _():
            @pl.when(jn >= NBUF)
            def _():
                wait_scatter(jn - NBUF, bn)
            gather(jn, bn)

        return carry

    lax.fori_loop(0, NCHUNK, body, 0)
    for j in range(NCHUNK - NBUF, NCHUNK):
        wait_scatter(j, j % NBUF)
    t0 = NCHUNK * CHUNK
    pltpu.async_copy(y_sp.at[src_v.at[pl.ds(t0, TAIL)]],
                     rows_v.at[0, pl.ds(0, TAIL)], gsem.at[0])
    pltpu.make_async_copy(y_sp.at[src_v.at[pl.ds(t0, TAIL)]],
                          rows_v.at[0, pl.ds(0, TAIL)], gsem.at[0]).wait()
    pltpu.sync_copy(rows_v.at[0, pl.ds(0, TAIL)],
                    acc_sp.at[dst_v.at[pl.ds(t0, TAIL)]], add=True)


def _msg_body(y_hbm, edge_hbm, out_hbm,
              src_v, dst_v, rows_v, zbuf_v, y_sp, acc_sp, gsem, ssem):
    c = lax.axis_index("c")
    s = lax.axis_index("s")
    wid = c * NS + s
    r0 = s * RPT
    pltpu.sync_copy(y_hbm.at[pl.ds(r0, RPT)], y_sp.at[pl.ds(r0, RPT)])
    _zero_acc(zbuf_v, acc_sp, r0)
    pltpu.sync_copy(edge_hbm.at[0, pl.ds(wid * EW, EW)], src_v)
    pltpu.sync_copy(edge_hbm.at[1, pl.ds(wid * EW, EW)], dst_v)
    plsc.subcore_barrier()
    _edge_loop(src_v, dst_v, rows_v, y_sp, acc_sp, gsem, ssem)
    plsc.subcore_barrier()
    pltpu.sync_copy(acc_sp.at[pl.ds(r0, RPT)], out_hbm.at[c, pl.ds(r0, RPT)])


@jax.jit
def _msg_kernel(y, edge_index):
    return pl.kernel(
        _msg_body,
        out_type=jax.ShapeDtypeStruct((NC, N, H), jnp.float32),
        mesh=_mesh,
        scratch_types=[
            pltpu.VMEM((EW,), jnp.int32),
            pltpu.VMEM((EW,), jnp.int32),
            pltpu.VMEM((NBUF, CHUNK, H), jnp.float32),
            pltpu.VMEM((ZR, H), jnp.float32),
            pltpu.VMEM_SHARED((N, H), jnp.float32),
            pltpu.VMEM_SHARED((N, H), jnp.float32),
            pltpu.SemaphoreType.DMA((NBUF,)),
            pltpu.SemaphoreType.DMA((NBUF,)),
        ],
        compiler_params=_sc_params,
    )(y, edge_index)


BN = 2048


def _prep_body(deg_ref, x_ref, w1_ref, dinv_ref, y1_ref):
    deg = jnp.sum(deg_ref[...], axis=0) + 1.0
    dinv = lax.rsqrt(deg)[:, None]
    dinv_ref[...] = dinv
    h = jnp.dot(x_ref[...], w1_ref[...], preferred_element_type=jnp.float32)
    y1_ref[...] = dinv * h


@jax.jit
def _prep_kernel(deg_parts, x, W1):
    return pl.pallas_call(
        _prep_body,
        grid=(pl.cdiv(N, BN),),
        in_specs=[
            pl.BlockSpec((NW, BN), lambda i: (0, i)),
            pl.BlockSpec((BN, D), lambda i: (i, 0)),
            pl.BlockSpec((D, H), lambda i: (0, 0)),
        ],
        out_specs=[
            pl.BlockSpec((BN, 1), lambda i: (i, 0)),
            pl.BlockSpec((BN, H), lambda i: (i, 0)),
        ],
        out_shape=[
            jax.ShapeDtypeStruct((N, 1), jnp.float32),
            jax.ShapeDtypeStruct((N, H), jnp.float32),
        ],
    )(deg_parts, x, W1)


def _mid_body(acc_ref, y1_ref, dinv_ref, w2_ref, b1_ref, y2_ref):
    a = acc_ref[0] + acc_ref[1] + y1_ref[...]
    dinv = dinv_ref[...]
    out1 = jnp.maximum(dinv * a + b1_ref[...], 0.0)
    h2 = jnp.dot(out1, w2_ref[...], preferred_element_type=jnp.float32)
    y2_ref[...] = dinv * h2


@jax.jit
def _mid_kernel(acc1, y1, dinv, W2p, b1r):
    return pl.pallas_call(
        _mid_body,
        grid=(pl.cdiv(N, BN),),
        in_specs=[
            pl.BlockSpec((NC, BN, H), lambda i: (0, i, 0)),
            pl.BlockSpec((BN, H), lambda i: (i, 0)),
            pl.BlockSpec((BN, 1), lambda i: (i, 0)),
            pl.BlockSpec((H, H), lambda i: (0, 0)),
            pl.BlockSpec((1, H), lambda i: (0, 0)),
        ],
        out_specs=pl.BlockSpec((BN, H), lambda i: (i, 0)),
        out_shape=jax.ShapeDtypeStruct((N, H), jnp.float32),
    )(acc1, y1, dinv, W2p, b1r)


def _fin_body(acc_ref, y2_ref, dinv_ref, b2_ref, out_ref):
    a = acc_ref[0] + acc_ref[1] + y2_ref[...]
    t = dinv_ref[...] * a + b2_ref[...]
    logits = t[:, :C]
    m = jnp.max(logits, axis=1, keepdims=True)
    ex = jnp.exp(logits - m)
    lse = jnp.log(jnp.sum(ex, axis=1, keepdims=True)) + m
    out_ref[...] = logits - lse


@jax.jit
def _fin_kernel(acc2, y2, dinv, b2p):
    return pl.pallas_call(
        _fin_body,
        grid=(pl.cdiv(N, BN),),
        in_specs=[
            pl.BlockSpec((NC, BN, H), lambda i: (0, i, 0)),
            pl.BlockSpec((BN, H), lambda i: (i, 0)),
            pl.BlockSpec((BN, 1), lambda i: (i, 0)),
            pl.BlockSpec((1, H), lambda i: (0, 0)),
        ],
        out_specs=pl.BlockSpec((BN, C), lambda i: (i, 0)),
        out_shape=jax.ShapeDtypeStruct((N, C), jnp.float32),
    )(acc2, y2, dinv, b2p)


def kernel(x, edge_index, W1, b1, W2, b2):
    W2p = jnp.pad(W2, ((0, 0), (0, H - C)))
    b1r = b1.reshape(1, H)
    b2p = jnp.pad(b2, (0, H - C)).reshape(1, H)

    deg_parts = _deg_kernel(edge_index)
    dinv, y1 = _prep_kernel(deg_parts, x, W1)
    acc1 = _msg_kernel(y1, edge_index)
    y2 = _mid_kernel(acc1, y1, dinv, W2p, b1r)
    acc2 = _msg_kernel(y2, edge_index)
    return _fin_kernel(acc2, y2, dinv, b2p)


# speedup vs baseline: 1.1686x; 1.0038x over previous
import jax
import jax.numpy as jnp
from jax import lax
from jax.experimental import pallas as pl
from jax.experimental.pallas import tpu as pltpu
from jax.experimental.pallas import tpu_sc as plsc

N = 10000
E = 320000
D = 128
H = 16
C = 7

NC = 2
NS = 16
NW = NC * NS
RPT = N // NS
EW = E // NW
CHUNK = 128
NCHUNK = EW // CHUNK
TAIL = EW - NCHUNK * CHUNK
NBUF = 8
PREF = 4
ZR = 128
NBLK = RPT // 16

_mesh = plsc.VectorSubcoreMesh(
    core_axis_name="c", subcore_axis_name="s", num_cores=NC, num_subcores=NS
)
_sc_params = pltpu.CompilerParams(
    needs_layout_passes=False, use_tc_tiling_on_sc=False)


def _worker_id():
    return lax.axis_index("c") * NS + lax.axis_index("s")


def _deg_body(edge_hbm, out_hbm, dst_v, deg_v):
    wid = _worker_id()
    pltpu.sync_copy(edge_hbm.at[1, pl.ds(wid * EW, EW)], dst_v)
    zero = jnp.zeros((16,), jnp.float32)

    def zbody(i, carry):
        deg_v[pl.ds(i * 16, 16)] = zero
        return carry

    lax.fori_loop(0, N // 16, zbody, 0)
    ones = jnp.full((16,), 1.0, jnp.float32)

    def body(i, carry):
        for u in range(5):
            idx = dst_v[pl.ds(i * 80 + u * 16, 16)]
            plsc.addupdate_scatter(deg_v, [idx], ones)
        return carry

    lax.fori_loop(0, EW // 80, body, 0)
    pltpu.sync_copy(deg_v, out_hbm.at[wid])


@jax.jit
def _deg_kernel(edge_index):
    return pl.kernel(
        _deg_body,
        out_type=jax.ShapeDtypeStruct((NW, N), jnp.float32),
        mesh=_mesh,
        scratch_types=[
            pltpu.VMEM((EW,), jnp.int32),
            pltpu.VMEM((N,), jnp.float32),
        ],
        compiler_params=_sc_params,
    )(edge_index)


def _zero_acc(zbuf_v, acc_sp, r0):
    zero = jnp.zeros((16,), jnp.float32)

    def zbody(i, carry):
        zbuf_v[i] = zero
        return carry

    lax.fori_loop(0, ZR, zbody, 0)
    for q in range(4):
        pltpu.sync_copy(zbuf_v, acc_sp.at[pl.ds(r0 + q * ZR, ZR)])
    pltpu.sync_copy(zbuf_v.at[pl.ds(0, RPT - 4 * ZR)],
                    acc_sp.at[pl.ds(r0 + 4 * ZR, RPT - 4 * ZR)])


def _edge_loop(src_v, dst_v, rows_v, y_sp, acc_sp, gsem, ssem):
    def gather(j, b):
        pltpu.async_copy(y_sp.at[src_v.at[pl.ds(j * CHUNK, CHUNK)]],
                         rows_v.at[b], gsem.at[b])

    def scatter(j, b):
        pltpu.async_copy(rows_v.at[b],
                         acc_sp.at[dst_v.at[pl.ds(j * CHUNK, CHUNK)]],
                         ssem.at[b], add=True)

    def wait_gather(j, b):
        pltpu.make_async_copy(y_sp.at[src_v.at[pl.ds(j * CHUNK, CHUNK)]],
                              rows_v.at[b], gsem.at[b]).wait()

    def wait_scatter(j, b):
        pltpu.make_async_copy(rows_v.at[b],
                              acc_sp.at[dst_v.at[pl.ds(j * CHUNK, CHUNK)]],
                              ssem.at[b]).wait()

    for jp in range(PREF):
        gather(jp, jp % NBUF)

    def body(j, carry):
        b = lax.rem(j, NBUF)
        wait_gather(j, b)
        scatter(j, b)
        jn = j + PREF
        bn = lax.rem(jn, NBUF)

        @pl.when(jn < NCHUNK)
        def _():
            @pl.when(jn >= NBUF)
            def _():
                wait_scatter(jn - NBUF, bn)
            gather(jn, bn)

        return carry

    lax.fori_loop(0, NCHUNK, body, 0)
    for j in range(NCHUNK - NBUF, NCHUNK):
        wait_scatter(j, j % NBUF)
    t0 = NCHUNK * CHUNK
    pltpu.async_copy(y_sp.at[src_v.at[pl.ds(t0, TAIL)]],
                     rows_v.at[0, pl.ds(0, TAIL)], gsem.at[0])
    pltpu.make_async_copy(y_sp.at[src_v.at[pl.ds(t0, TAIL)]],
                          rows_v.at[0, pl.ds(0, TAIL)], gsem.at[0]).wait()
    pltpu.sync_copy(rows_v.at[0, pl.ds(0, TAIL)],
                    acc_sp.at[dst_v.at[pl.ds(t0, TAIL)]], add=True)


def _msg_body(y_hbm, edge_hbm, out_hbm,
              src_v, dst_v, rows_v, zbuf_v, y_sp, acc_sp, gsem, ssem):
    c = lax.axis_index("c")
    s = lax.axis_index("s")
    wid = c * NS + s
    r0 = s * RPT
    pltpu.sync_copy(y_hbm.at[pl.ds(r0, RPT)], y_sp.at[pl.ds(r0, RPT)])
    _zero_acc(zbuf_v, acc_sp, r0)
    pltpu.sync_copy(edge_hbm.at[0, pl.ds(wid * EW, EW)], src_v)
    pltpu.sync_copy(edge_hbm.at[1, pl.ds(wid * EW, EW)], dst_v)
    plsc.subcore_barrier()
    _edge_loop(src_v, dst_v, rows_v, y_sp, acc_sp, gsem, ssem)
    plsc.subcore_barrier()
    pltpu.sync_copy(acc_sp.at[pl.ds(r0, RPT)], out_hbm.at[c, pl.ds(r0, RPT)])


@jax.jit
def _msg_kernel(y, edge_index):
    return pl.kernel(
        _msg_body,
        out_type=jax.ShapeDtypeStruct((NC, N, H), jnp.float32),
        mesh=_mesh,
        scratch_types=[
            pltpu.VMEM((EW,), jnp.int32),
            pltpu.VMEM((EW,), jnp.int32),
            pltpu.VMEM((NBUF, CHUNK, H), jnp.float32),
            pltpu.VMEM((ZR, H), jnp.float32),
            pltpu.VMEM_SHARED((N, H), jnp.float32),
            pltpu.VMEM_SHARED((N, H), jnp.float32),
            pltpu.SemaphoreType.DMA((NBUF,)),
            pltpu.SemaphoreType.DMA((NBUF,)),
        ],
        compiler_params=_sc_params,
    )(y, edge_index)


BN = 2048


def _prep_body(deg_ref, x_ref, w1_ref, dinv_ref, y1_ref):
    deg = jnp.sum(deg_ref[...], axis=0) + 1.0
    dinv = lax.rsqrt(deg)[:, None]
    dinv_ref[...] = dinv
    h = jnp.dot(x_ref[...], w1_ref[...], preferred_element_type=jnp.float32)
    y1_ref[...] = dinv * h


@jax.jit
def _prep_kernel(deg_parts, x, W1):
    return pl.pallas_call(
        _prep_body,
        grid=(pl.cdiv(N, BN),),
        in_specs=[
            pl.BlockSpec((NW, BN), lambda i: (0, i)),
            pl.BlockSpec((BN, D), lambda i: (i, 0)),
            pl.BlockSpec((D, H), lambda i: (0, 0)),
        ],
        out_specs=[
            pl.BlockSpec((BN, 1), lambda i: (i, 0)),
            pl.BlockSpec((BN, H), lambda i: (i, 0)),
        ],
        out_shape=[
            jax.ShapeDtypeStruct((N, 1), jnp.float32),
            jax.ShapeDtypeStruct((N, H), jnp.float32),
        ],
    )(deg_parts, x, W1)


def _mid_body(acc_ref, y1_ref, dinv_ref, w2_ref, b1_ref, y2_ref):
    a = acc_ref[0] + acc_ref[1] + y1_ref[...]
    dinv = dinv_ref[...]
    out1 = jnp.maximum(dinv * a + b1_ref[...], 0.0)
    h2 = jnp.dot(out1, w2_ref[...], preferred_element_type=jnp.float32)
    y2_ref[...] = dinv * h2


@jax.jit
def _mid_kernel(acc1, y1, dinv, W2p, b1r):
    return pl.pallas_call(
        _mid_body,
        grid=(pl.cdiv(N, BN),),
        in_specs=[
            pl.BlockSpec((NC, BN, H), lambda i: (0, i, 0)),
            pl.BlockSpec((BN, H), lambda i: (i, 0)),
            pl.BlockSpec((BN, 1), lambda i: (i, 0)),
            pl.BlockSpec((H, H), lambda i: (0, 0)),
            pl.BlockSpec((1, H), lambda i: (0, 0)),
        ],
        out_specs=pl.BlockSpec((BN, H), lambda i: (i, 0)),
        out_shape=jax.ShapeDtypeStruct((N, H), jnp.float32),
    )(acc1, y1, dinv, W2p, b1r)


def _fin_body(acc_ref, y2_ref, dinv_ref, b2_ref, out_ref):
    a = acc_ref[0] + acc_ref[1] + y2_ref[...]
    t = dinv_ref[...] * a + b2_ref[...]
    logits = t[:, :C]
    m = jnp.max(logits, axis=1, keepdims=True)
    ex = jnp.exp(logits - m)
    lse = jnp.log(jnp.sum(ex, axis=1, keepdims=True)) + m
    out_ref[...] = logits - lse


@jax.jit
def _fin_kernel(acc2, y2, dinv, b2p):
    return pl.pallas_call(
        _fin_body,
        grid=(pl.cdiv(N, BN),),
        in_specs=[
            pl.BlockSpec((NC, BN, H), lambda i: (0, i, 0)),
            pl.BlockSpec((BN, H), lambda i: (i, 0)),
            pl.BlockSpec((BN, 1), lambda i: (i, 0)),
            pl.BlockSpec((1, H), lambda i: (0, 0)),
        ],
        out_specs=pl.BlockSpec((BN, C), lambda i: (i, 0)),
        out_shape=jax.ShapeDtypeStruct((N, C), jnp.float32),
    )(acc2, y2, dinv, b2p)


def kernel(x, edge_index, W1, b1, W2, b2):
    W2p = jnp.pad(W2, ((0, 0), (0, H - C)))
    b1r = b1.reshape(1, H)
    b2p = jnp.pad(b2, (0, H - C)).reshape(1, H)

    deg_parts = _deg_kernel(edge_index)
    dinv, y1 = _prep_kernel(deg_parts, x, W1)
    acc1 = _msg_kernel(y1, edge_index)
    y2 = _mid_kernel(acc1, y1, dinv, W2p, b1r)
    acc2 = _msg_kernel(y2, edge_index)
    return _fin_kernel(acc2, y2, dinv, b2p)


# speedup vs baseline: 1.1712x; 1.0022x over previous
import jax
import jax.numpy as jnp
from jax import lax
from jax.experimental import pallas as pl
from jax.experimental.pallas import tpu as pltpu
from jax.experimental.pallas import tpu_sc as plsc

N = 10000
E = 320000
D = 128
H = 16
C = 7

NC = 2
NS = 16
NW = NC * NS
RPT = N // NS
EW = E // NW
CHUNK = 128
NCHUNK = EW // CHUNK
TAIL = EW - NCHUNK * CHUNK
NBUF = 8
PREF = 4
ZR = 128
NBLK = RPT // 16

_mesh = plsc.VectorSubcoreMesh(
    core_axis_name="c", subcore_axis_name="s", num_cores=NC, num_subcores=NS
)
_sc_params = pltpu.CompilerParams(
    needs_layout_passes=False, use_tc_tiling_on_sc=False)


def _worker_id():
    return lax.axis_index("c") * NS + lax.axis_index("s")


def _deg_body(edge_hbm, out_hbm, dst_v, deg_v):
    wid = _worker_id()
    pltpu.sync_copy(edge_hbm.at[1, pl.ds(wid * EW, EW)], dst_v)
    zero = jnp.zeros((16,), jnp.float32)

    def zbody(i, carry):
        deg_v[pl.ds(i * 16, 16)] = zero
        return carry

    lax.fori_loop(0, N // 16, zbody, 0)
    ones = jnp.full((16,), 1.0, jnp.float32)

    def body(i, carry):
        for u in range(5):
            idx = dst_v[pl.ds(i * 80 + u * 16, 16)]
            plsc.addupdate_scatter(deg_v, [idx], ones)
        return carry

    lax.fori_loop(0, EW // 80, body, 0)
    pltpu.sync_copy(deg_v, out_hbm.at[wid])


@jax.jit
def _deg_kernel(edge_index):
    return pl.kernel(
        _deg_body,
        out_type=jax.ShapeDtypeStruct((NW, N), jnp.float32),
        mesh=_mesh,
        scratch_types=[
            pltpu.VMEM((EW,), jnp.int32),
            pltpu.VMEM((N,), jnp.float32),
        ],
        compiler_params=_sc_params,
    )(edge_index)


def _zero_acc(zbuf_v, acc_sp, r0):
    zero = jnp.zeros((16,), jnp.float32)

    def zbody(i, carry):
        zbuf_v[i] = zero
        return carry

    lax.fori_loop(0, ZR, zbody, 0)
    for q in range(4):
        pltpu.sync_copy(zbuf_v, acc_sp.at[pl.ds(r0 + q * ZR, ZR)])
    pltpu.sync_copy(zbuf_v.at[pl.ds(0, RPT - 4 * ZR)],
                    acc_sp.at[pl.ds(r0 + 4 * ZR, RPT - 4 * ZR)])


def _edge_loop(src_v, dst_v, rows_v, y_sp, acc_sp, gsem, ssem):
    def gather(j, b):
        pltpu.async_copy(y_sp.at[src_v.at[pl.ds(j * CHUNK, CHUNK)]],
                         rows_v.at[b], gsem.at[b])

    def scatter(j, b):
        pltpu.async_copy(rows_v.at[b],
                         acc_sp.at[dst_v.at[pl.ds(j * CHUNK, CHUNK)]],
                         ssem.at[b], add=True)

    def wait_gather(j, b):
        pltpu.make_async_copy(y_sp.at[src_v.at[pl.ds(j * CHUNK, CHUNK)]],
                              rows_v.at[b], gsem.at[b]).wait()

    def wait_scatter(j, b):
        pltpu.make_async_copy(rows_v.at[b],
                              acc_sp.at[dst_v.at[pl.ds(j * CHUNK, CHUNK)]],
                              ssem.at[b]).wait()

    for jp in range(PREF):
        gather(jp, jp % NBUF)

    def body(j, carry):
        b = lax.rem(j, NBUF)
        wait_gather(j, b)
        scatter(j, b)
        jn = j + PREF
        bn = lax.rem(jn, NBUF)

        @pl.when(jn < NCHUNK)
        def _():
            @pl.when(jn >= NBUF)
            def _():
                wait_scatter(jn - NBUF, bn)
            gather(jn, bn)

        return carry

    lax.fori_loop(0, NCHUNK, body, 0)
    for j in range(NCHUNK - NBUF, NCHUNK):
        wait_scatter(j, j % NBUF)
    t0 = NCHUNK * CHUNK
    pltpu.async_copy(y_sp.at[src_v.at[pl.ds(t0, TAIL)]],
                     rows_v.at[0, pl.ds(0, TAIL)], gsem.at[0])
    pltpu.make_async_copy(y_sp.at[src_v.at[pl.ds(t0, TAIL)]],
                          rows_v.at[0, pl.ds(0, TAIL)], gsem.at[0]).wait()
    pltpu.sync_copy(rows_v.at[0, pl.ds(0, TAIL)],
                    acc_sp.at[dst_v.at[pl.ds(t0, TAIL)]], add=True)


def _msg_body(y_hbm, edge_hbm, out_hbm,
              src_v, dst_v, rows_v, zbuf_v, y_sp, acc_sp, gsem, ssem):
    c = lax.axis_index("c")
    s = lax.axis_index("s")
    wid = c * NS + s
    r0 = s * RPT
    pltpu.sync_copy(y_hbm.at[pl.ds(r0, RPT)], y_sp.at[pl.ds(r0, RPT)])
    _zero_acc(zbuf_v, acc_sp, r0)
    pltpu.sync_copy(edge_hbm.at[0, pl.ds(wid * EW, EW)], src_v)
    pltpu.sync_copy(edge_hbm.at[1, pl.ds(wid * EW, EW)], dst_v)
    plsc.subcore_barrier()
    _edge_loop(src_v, dst_v, rows_v, y_sp, acc_sp, gsem, ssem)
    plsc.subcore_barrier()
    pltpu.sync_copy(acc_sp.at[pl.ds(r0, RPT)], out_hbm.at[c, pl.ds(r0, RPT)])


@jax.jit
def _msg_kernel(y, edge_index):
    return pl.kernel(
        _msg_body,
        out_type=jax.ShapeDtypeStruct((NC, N, H), jnp.float32),
        mesh=_mesh,
        scratch_types=[
            pltpu.VMEM((EW,), jnp.int32),
            pltpu.VMEM((EW,), jnp.int32),
            pltpu.VMEM((NBUF, CHUNK, H), jnp.float32),
            pltpu.VMEM((ZR, H), jnp.float32),
            pltpu.VMEM_SHARED((N, H), jnp.float32),
            pltpu.VMEM_SHARED((N, H), jnp.float32),
            pltpu.SemaphoreType.DMA((NBUF,)),
            pltpu.SemaphoreType.DMA((NBUF,)),
        ],
        compiler_params=_sc_params,
    )(y, edge_index)


BN = 2048


def _h1_body(x_ref, w1_ref, h1_ref):
    h1_ref[...] = jnp.dot(x_ref[...], w1_ref[...],
                          preferred_element_type=jnp.float32)


@jax.jit
def _h1_kernel(x, W1):
    return pl.pallas_call(
        _h1_body,
        grid=(pl.cdiv(N, BN),),
        in_specs=[
            pl.BlockSpec((BN, D), lambda i: (i, 0)),
            pl.BlockSpec((D, H), lambda i: (0, 0)),
        ],
        out_specs=pl.BlockSpec((BN, H), lambda i: (i, 0)),
        out_shape=jax.ShapeDtypeStruct((N, H), jnp.float32),
    )(x, W1)


def _prep_body(deg_ref, h1_ref, dinv_ref, y1_ref):
    deg = jnp.sum(deg_ref[...], axis=0) + 1.0
    dinv = lax.rsqrt(deg)[:, None]
    dinv_ref[...] = dinv
    y1_ref[...] = dinv * h1_ref[...]


@jax.jit
def _prep_kernel(deg_parts, h1):
    return pl.pallas_call(
        _prep_body,
        grid=(pl.cdiv(N, BN),),
        in_specs=[
            pl.BlockSpec((NW, BN), lambda i: (0, i)),
            pl.BlockSpec((BN, H), lambda i: (i, 0)),
        ],
        out_specs=[
            pl.BlockSpec((BN, 1), lambda i: (i, 0)),
            pl.BlockSpec((BN, H), lambda i: (i, 0)),
        ],
        out_shape=[
            jax.ShapeDtypeStruct((N, 1), jnp.float32),
            jax.ShapeDtypeStruct((N, H), jnp.float32),
        ],
    )(deg_parts, h1)


def _mid_body(acc_ref, y1_ref, dinv_ref, w2_ref, b1_ref, y2_ref):
    a = acc_ref[0] + acc_ref[1] + y1_ref[...]
    dinv = dinv_ref[...]
    out1 = jnp.maximum(dinv * a + b1_ref[...], 0.0)
    h2 = jnp.dot(out1, w2_ref[...], preferred_element_type=jnp.float32)
    y2_ref[...] = dinv * h2


@jax.jit
def _mid_kernel(acc1, y1, dinv, W2p, b1r):
    return pl.pallas_call(
        _mid_body,
        grid=(pl.cdiv(N, BN),),
        in_specs=[
            pl.BlockSpec((NC, BN, H), lambda i: (0, i, 0)),
            pl.BlockSpec((BN, H), lambda i: (i, 0)),
            pl.BlockSpec((BN, 1), lambda i: (i, 0)),
            pl.BlockSpec((H, H), lambda i: (0, 0)),
            pl.BlockSpec((1, H), lambda i: (0, 0)),
        ],
        out_specs=pl.BlockSpec((BN, H), lambda i: (i, 0)),
        out_shape=jax.ShapeDtypeStruct((N, H), jnp.float32),
    )(acc1, y1, dinv, W2p, b1r)


def _fin_body(acc_ref, y2_ref, dinv_ref, b2_ref, out_ref):
    a = acc_ref[0] + acc_ref[1] + y2_ref[...]
    t = dinv_ref[...] * a + b2_ref[...]
    logits = t[:, :C]
    m = jnp.max(logits, axis=1, keepdims=True)
    ex = jnp.exp(logits - m)
    lse = jnp.log(jnp.sum(ex, axis=1, keepdims=True)) + m
    out_ref[...] = logits - lse


@jax.jit
def _fin_kernel(acc2, y2, dinv, b2p):
    return pl.pallas_call(
        _fin_body,
        grid=(pl.cdiv(N, BN),),
        in_specs=[
            pl.BlockSpec((NC, BN, H), lambda i: (0, i, 0)),
            pl.BlockSpec((BN, H), lambda i: (i, 0)),
            pl.BlockSpec((BN, 1), lambda i: (i, 0)),
            pl.BlockSpec((1, H), lambda i: (0, 0)),
        ],
        out_specs=pl.BlockSpec((BN, C), lambda i: (i, 0)),
        out_shape=jax.ShapeDtypeStruct((N, C), jnp.float32),
    )(acc2, y2, dinv, b2p)


def kernel(x, edge_index, W1, b1, W2, b2):
    W2p = jnp.pad(W2, ((0, 0), (0, H - C)))
    b1r = b1.reshape(1, H)
    b2p = jnp.pad(b2, (0, H - C)).reshape(1, H)

    deg_parts = _deg_kernel(edge_index)
    h1 = _h1_kernel(x, W1)
    dinv, y1 = _prep_kernel(deg_parts, h1)
    acc1 = _msg_kernel(y1, edge_index)
    y2 = _mid_kernel(acc1, y1, dinv, W2p, b1r)
    acc2 = _msg_kernel(y2, edge_index)
    return _fin_kernel(acc2, y2, dinv, b2p)


# speedup vs baseline: 1.2708x; 1.0850x over previous
import jax
import jax.numpy as jnp
from jax import lax
from jax.experimental import pallas as pl
from jax.experimental.pallas import tpu as pltpu
from jax.experimental.pallas import tpu_sc as plsc

N = 10000
E = 320000
D = 128
H = 16
C = 7

NC = 2
NS = 16
NW = NC * NS
RPT = N // NS
EW = E // NW
CHUNK = 128
NCHUNK = EW // CHUNK
TAIL = EW - NCHUNK * CHUNK
NBUF = 8
PREF = 4
ZR = 128
NBLK = RPT // 16

_mesh = plsc.VectorSubcoreMesh(
    core_axis_name="c", subcore_axis_name="s", num_cores=NC, num_subcores=NS
)
_sc_params = pltpu.CompilerParams(
    needs_layout_passes=False, use_tc_tiling_on_sc=False)


def _worker_id():
    return lax.axis_index("c") * NS + lax.axis_index("s")


def _deg_body(edge_hbm, out_hbm, dst_v, deg_v):
    wid = _worker_id()
    pltpu.sync_copy(edge_hbm.at[1, pl.ds(wid * EW, EW)], dst_v)
    zero = jnp.zeros((16,), jnp.float32)

    def zbody(i, carry):
        deg_v[pl.ds(i * 16, 16)] = zero
        return carry

    lax.fori_loop(0, N // 16, zbody, 0)
    ones = jnp.full((16,), 1.0, jnp.float32)

    def body(i, carry):
        for u in range(5):
            idx = dst_v[pl.ds(i * 80 + u * 16, 16)]
            plsc.addupdate_scatter(deg_v, [idx], ones)
        return carry

    lax.fori_loop(0, EW // 80, body, 0)
    pltpu.sync_copy(deg_v, out_hbm.at[wid])


@jax.jit
def _deg_kernel(edge_index):
    return pl.kernel(
        _deg_body,
        out_type=jax.ShapeDtypeStruct((NW, N), jnp.float32),
        mesh=_mesh,
        scratch_types=[
            pltpu.VMEM((EW,), jnp.int32),
            pltpu.VMEM((N,), jnp.float32),
        ],
        compiler_params=_sc_params,
    )(edge_index)


def _zero_acc(zbuf_v, acc_sp, r0):
    zero = jnp.zeros((16,), jnp.float32)

    def zbody(i, carry):
        zbuf_v[i] = zero
        return carry

    lax.fori_loop(0, ZR, zbody, 0)
    for q in range(4):
        pltpu.sync_copy(zbuf_v, acc_sp.at[pl.ds(r0 + q * ZR, ZR)])
    pltpu.sync_copy(zbuf_v.at[pl.ds(0, RPT - 4 * ZR)],
                    acc_sp.at[pl.ds(r0 + 4 * ZR, RPT - 4 * ZR)])


def _edge_loop(src_v, dst_v, rows_v, y_sp, acc_sp, gsem, ssem):
    def gather(j, b):
        pltpu.async_copy(y_sp.at[src_v.at[pl.ds(j * CHUNK, CHUNK)]],
                         rows_v.at[b], gsem.at[b])

    def scatter(j, b):
        pltpu.async_copy(rows_v.at[b],
                         acc_sp.at[dst_v.at[pl.ds(j * CHUNK, CHUNK)]],
                         ssem.at[b], add=True)

    def wait_gather(j, b):
        pltpu.make_async_copy(y_sp.at[src_v.at[pl.ds(j * CHUNK, CHUNK)]],
                              rows_v.at[b], gsem.at[b]).wait()

    def wait_scatter(j, b):
        pltpu.make_async_copy(rows_v.at[b],
                              acc_sp.at[dst_v.at[pl.ds(j * CHUNK, CHUNK)]],
                              ssem.at[b]).wait()

    for jp in range(PREF):
        gather(jp, jp % NBUF)

    def body(j, carry):
        b = lax.rem(j, NBUF)
        wait_gather(j, b)
        scatter(j, b)
        jn = j + PREF
        bn = lax.rem(jn, NBUF)

        @pl.when(jn < NCHUNK)
        def _():
            @pl.when(jn >= NBUF)
            def _():
                wait_scatter(jn - NBUF, bn)
            gather(jn, bn)

        return carry

    lax.fori_loop(0, NCHUNK, body, 0)
    for j in range(NCHUNK - NBUF, NCHUNK):
        wait_scatter(j, j % NBUF)
    t0 = NCHUNK * CHUNK
    pltpu.async_copy(y_sp.at[src_v.at[pl.ds(t0, TAIL)]],
                     rows_v.at[0, pl.ds(0, TAIL)], gsem.at[0])
    pltpu.make_async_copy(y_sp.at[src_v.at[pl.ds(t0, TAIL)]],
                          rows_v.at[0, pl.ds(0, TAIL)], gsem.at[0]).wait()
    pltpu.sync_copy(rows_v.at[0, pl.ds(0, TAIL)],
                    acc_sp.at[dst_v.at[pl.ds(t0, TAIL)]], add=True)


def _msg_body(y_hbm, edge_hbm, out_hbm,
              src_v, dst_v, rows_v, zbuf_v, y_sp, acc_sp, gsem, ssem):
    c = lax.axis_index("c")
    s = lax.axis_index("s")
    wid = c * NS + s
    r0 = s * RPT
    pltpu.sync_copy(y_hbm.at[pl.ds(r0, RPT)], y_sp.at[pl.ds(r0, RPT)])
    _zero_acc(zbuf_v, acc_sp, r0)
    pltpu.sync_copy(edge_hbm.at[0, pl.ds(wid * EW, EW)], src_v)
    pltpu.sync_copy(edge_hbm.at[1, pl.ds(wid * EW, EW)], dst_v)
    plsc.subcore_barrier()
    _edge_loop(src_v, dst_v, rows_v, y_sp, acc_sp, gsem, ssem)
    plsc.subcore_barrier()
    pltpu.sync_copy(acc_sp.at[pl.ds(r0, RPT)], out_hbm.at[c, pl.ds(r0, RPT)])


@jax.jit
def _msg_kernel(y, edge_index):
    return pl.kernel(
        _msg_body,
        out_type=jax.ShapeDtypeStruct((NC, N, H), jnp.float32),
        mesh=_mesh,
        scratch_types=[
            pltpu.VMEM((EW,), jnp.int32),
            pltpu.VMEM((EW,), jnp.int32),
            pltpu.VMEM((NBUF, CHUNK, H), jnp.float32),
            pltpu.VMEM((ZR, H), jnp.float32),
            pltpu.VMEM_SHARED((N, H), jnp.float32),
            pltpu.VMEM_SHARED((N, H), jnp.float32),
            pltpu.SemaphoreType.DMA((NBUF,)),
            pltpu.SemaphoreType.DMA((NBUF,)),
        ],
        compiler_params=_sc_params,
    )(y, edge_index)


BN = 2048


def _h1_body(x_ref, w1_ref, h1_ref):
    h1_ref[...] = jnp.dot(x_ref[...], w1_ref[...],
                          preferred_element_type=jnp.float32)


@jax.jit
def _h1_kernel(x, W1):
    return pl.pallas_call(
        _h1_body,
        grid=(pl.cdiv(N, BN),),
        in_specs=[
            pl.BlockSpec((BN, D), lambda i: (i, 0)),
            pl.BlockSpec((D, H), lambda i: (0, 0)),
        ],
        out_specs=pl.BlockSpec((BN, H), lambda i: (i, 0)),
        out_shape=jax.ShapeDtypeStruct((N, H), jnp.float32),
    )(x, W1)


def _prep_body(deg_ref, h1_ref, dinv_ref, y1_ref):
    deg = jnp.sum(deg_ref[...], axis=0) + 1.0
    dinv = lax.rsqrt(deg)[:, None]
    dinv_ref[...] = dinv
    y1_ref[...] = dinv * h1_ref[...]


@jax.jit
def _prep_kernel(deg_parts, h1):
    return pl.pallas_call(
        _prep_body,
        grid=(pl.cdiv(N, BN),),
        in_specs=[
            pl.BlockSpec((NW, BN), lambda i: (0, i)),
            pl.BlockSpec((BN, H), lambda i: (i, 0)),
        ],
        out_specs=[
            pl.BlockSpec((BN, 1), lambda i: (i, 0)),
            pl.BlockSpec((BN, H), lambda i: (i, 0)),
        ],
        out_shape=[
            jax.ShapeDtypeStruct((N, 1), jnp.float32),
            jax.ShapeDtypeStruct((N, H), jnp.float32),
        ],
    )(deg_parts, h1)


def _mid_body(acc_ref, y1_ref, dinv_ref, w2_ref, b1_ref, y2_ref):
    a = acc_ref[0] + acc_ref[1] + y1_ref[...]
    dinv = dinv_ref[...]
    out1 = jnp.maximum(dinv * a + b1_ref[...], 0.0)
    h2 = jnp.dot(out1, w2_ref[...], preferred_element_type=jnp.float32)
    y2_ref[...] = dinv * h2


@jax.jit
def _mid_kernel(acc1, y1, dinv, W2p, b1r):
    return pl.pallas_call(
        _mid_body,
        grid=(pl.cdiv(N, BN),),
        in_specs=[
            pl.BlockSpec((NC, BN, H), lambda i: (0, i, 0)),
            pl.BlockSpec((BN, H), lambda i: (i, 0)),
            pl.BlockSpec((BN, 1), lambda i: (i, 0)),
            pl.BlockSpec((H, H), lambda i: (0, 0)),
            pl.BlockSpec((1, H), lambda i: (0, 0)),
        ],
        out_specs=pl.BlockSpec((BN, H), lambda i: (i, 0)),
        out_shape=jax.ShapeDtypeStruct((N, H), jnp.float32),
    )(acc1, y1, dinv, W2p, b1r)


NF = N * H // 128


def _finf_body(acc_ref, y2f_ref, dinvB_ref, b2B_ref, mm_ref, ss_ref,
               mask_ref, out_ref):
    t = dinvB_ref[...] * (acc_ref[0] + acc_ref[1] + y2f_ref[...]) + b2B_ref[...]
    mean = jnp.dot(t, mm_ref[...], preferred_element_type=jnp.float32)
    tc = t - mean
    e = jnp.exp(tc) * mask_ref[...]
    s = jnp.dot(e, ss_ref[...], preferred_element_type=jnp.float32)
    out_ref[...] = tc - jnp.log(s)


@jax.jit
def _finf_kernel(acc2f, y2f, dinvB, b2B, MmB, SB, maskB):
    return pl.pallas_call(
        _finf_body,
        grid=(1,),
        in_specs=[
            pl.BlockSpec((NC, NF, 128), lambda i: (0, 0, 0)),
            pl.BlockSpec((NF, 128), lambda i: (0, 0)),
            pl.BlockSpec((NF, 128), lambda i: (0, 0)),
            pl.BlockSpec((1, 128), lambda i: (0, 0)),
            pl.BlockSpec((128, 128), lambda i: (0, 0)),
            pl.BlockSpec((128, 128), lambda i: (0, 0)),
            pl.BlockSpec((1, 128), lambda i: (0, 0)),
        ],
        out_specs=pl.BlockSpec((NF, 128), lambda i: (0, 0)),
        out_shape=jax.ShapeDtypeStruct((NF, 128), jnp.float32),
    )(acc2f, y2f, dinvB, b2B, MmB, SB, maskB)


def kernel(x, edge_index, W1, b1, W2, b2):
    W2p = jnp.pad(W2, ((0, 0), (0, H - C)))
    b1r = b1.reshape(1, H)
    b2p = jnp.pad(b2, (0, H - C)).reshape(1, H)

    mask7 = (jnp.arange(H) < C).astype(jnp.float32)
    I8 = jnp.eye(8, dtype=jnp.float32)
    MmB = jnp.kron(I8, jnp.tile(mask7[:, None] / C, (1, H)))
    SB = jnp.kron(I8, jnp.tile(mask7[:, None], (1, H)))
    maskB = jnp.tile(mask7, 8)[None, :]
    b2B = jnp.tile(b2p[0], 8)[None, :]

    deg_parts = _deg_kernel(edge_index)
    h1 = _h1_kernel(x, W1)
    dinv, y1 = _prep_kernel(deg_parts, h1)
    acc1 = _msg_kernel(y1, edge_index)
    y2 = _mid_kernel(acc1, y1, dinv, W2p, b1r)
    acc2 = _msg_kernel(y2, edge_index)
    dinvB = jnp.broadcast_to(dinv, (N, H)).reshape(NF, 128)
    out_flat = _finf_kernel(acc2.reshape(NC, NF, 128), y2.reshape(NF, 128),
                            dinvB, b2B, MmB, SB, maskB)
    return out_flat.reshape(N, H)[:, :C]
